# baseline (device time: 71731 ns/iter reference)
import jax
import jax.numpy as jnp
from jax import lax
from jax.experimental import pallas as pl
from jax.experimental.pallas import tpu as pltpu

B, S, D = 2, 512, 2048
H, Dh, Dr = 16, 128, 32
DC = 128
M = B * S
SCALE = (Dh + Dr) ** -0.5
BF = jnp.bfloat16
F32 = jnp.float32
NJ = 4


LOG2E = 1.4426950408889634


def _proj_comm_body(x_ref, wdkv_ref, wuk_ref, wuv_ref, wkr_ref, wq_ref, wqr_ref,
                    q_ref, k_ref, v_ref, kr_ref,
                    xb_s, c_s, wuk_s, wuv_s, c_r, wuk_r, wuv_r,
                    send_sems, recv_sems):
    j = pl.program_id(0)
    my_x = lax.axis_index("x")
    my_y = lax.axis_index("y")
    my_z = lax.axis_index("z")
    nbr = (1 - my_x, my_y, my_z)

    pairs = [(c_s, c_r), (wuk_s, wuk_r), (wuv_s, wuv_r)]

    def mk(i, src, dst):
        return pltpu.make_async_remote_copy(
            src_ref=src, dst_ref=dst,
            send_sem=send_sems.at[i], recv_sem=recv_sems.at[i],
            device_id=nbr, device_id_type=pl.DeviceIdType.MESH,
        )

    @pl.when(j == 0)
    def _():
        barrier = pltpu.get_barrier_semaphore()
        pl.semaphore_signal(barrier, inc=1, device_id=nbr,
                            device_id_type=pl.DeviceIdType.MESH)
        xb = x_ref[...].reshape(M, D).astype(BF)
        xb_s[...] = xb
        c_s[...] = jnp.dot(
            xb, wdkv_ref[...].astype(BF), preferred_element_type=F32
        ).astype(BF)
        wuk_s[...] = wuk_ref[...].astype(BF)
        wuv_s[...] = wuv_ref[...].astype(BF)
        pl.semaphore_wait(barrier, 1)
        for i, (s, d) in enumerate(pairs):
            mk(i, s, d).start()

    xb = xb_s[...]
    qd = (
        jnp.dot(xb, wq_ref[...].astype(BF), preferred_element_type=F32)
        * (SCALE * LOG2E)
    ).astype(BF)
    qrd = (
        jnp.dot(xb, wqr_ref[...].astype(BF), preferred_element_type=F32)
        * (SCALE * LOG2E)
    ).astype(BF)
    zpad = jnp.zeros((M, Dh - Dr), BF)
    for i in range(4):
        q_ref[:, i * 256:i * 256 + Dh] = qd[:, i * Dh:(i + 1) * Dh]
        q_ref[:, i * 256 + Dh:(i + 1) * 256] = jnp.concatenate(
            [qrd[:, i * Dr:(i + 1) * Dr], zpad], axis=1
        )

    @pl.when(j == NJ - 1)
    def _():
        kr_ref[...] = jnp.dot(
            xb, wkr_ref[...].astype(BF), preferred_element_type=F32
        ).astype(BF)
        for i, (s, d) in enumerate(pairs):
            mk(i, s, d).wait()
        k = jnp.dot(c_s[...], wuk_s[...], preferred_element_type=F32)
        k += jnp.dot(c_r[...], wuk_r[...], preferred_element_type=F32)
        k_ref[...] = k.astype(BF)
        v = jnp.dot(c_s[...], wuv_s[...], preferred_element_type=F32)
        v += jnp.dot(c_r[...], wuv_r[...], preferred_element_type=F32)
        v_ref[...] = v.astype(BF)


GH = 4
NG = H // GH


def _attn_o_body(q_ref, k_ref, v_ref, kr_ref, wo_ref, out_ref,
                 o_s, wo_bf):
    i = pl.program_id(0)
    g = i % NG

    @pl.when(i == 0)
    def _():
        wo_bf[...] = wo_ref[...].astype(BF)

    nt = (((1,), (1,)), ((), ()))
    kr = kr_ref[...]
    for hh in range(GH):
        q = q_ref[:, hh * 256:(hh + 1) * 256]
        k = k_ref[:, hh * Dh:(hh + 1) * Dh]
        kcat = jnp.concatenate([k, kr, k[:, :Dh - Dr]], axis=1)
        s = lax.dot_general(q, kcat, nt, preferred_element_type=F32)
        p = jnp.exp2(s.astype(BF))
        r = 1.0 / jnp.sum(p.astype(F32), axis=-1, keepdims=True)
        o = jnp.dot(
            p, v_ref[:, hh * Dh:(hh + 1) * Dh],
            preferred_element_type=F32,
        )
        o_s[g, :, hh * Dh:(hh + 1) * Dh] = (o * r).astype(BF)

    @pl.when(g == NG - 1)
    def _():
        GW = GH * Dh
        out = jnp.dot(o_s[0], wo_bf[0 * GW:1 * GW, :],
                      preferred_element_type=F32)
        for gg in range(1, NG):
            out += jnp.dot(o_s[gg], wo_bf[gg * GW:(gg + 1) * GW, :],
                           preferred_element_type=F32)
        out_ref[...] = out.reshape(1, S, D)


def kernel(x, Wdkv, Wuk, Wuv, Wq, Wqr, Wkr, Wo):
    BD = D // NJ
    BR = H * Dr // NJ

    Q, K, V, Kr = pl.pallas_call(
        _proj_comm_body,
        grid=(NJ,),
        in_specs=[
            pl.BlockSpec((B, S, D), lambda j: (0, 0, 0)),
            pl.BlockSpec((D, DC), lambda j: (0, 0)),
            pl.BlockSpec((DC, D), lambda j: (0, 0)),
            pl.BlockSpec((DC, D), lambda j: (0, 0)),
            pl.BlockSpec((D, Dr), lambda j: (0, 0)),
            pl.BlockSpec((D, BD), lambda j: (0, j)),
            pl.BlockSpec((D, BR), lambda j: (0, j)),
        ],
        out_specs=[
            pl.BlockSpec((M, 4 * 256), lambda j: (0, j)),
            pl.BlockSpec((M, D), lambda j: (0, 0)),
            pl.BlockSpec((M, D), lambda j: (0, 0)),
            pl.BlockSpec((M, Dr), lambda j: (0, 0)),
        ],
        out_shape=[
            jax.ShapeDtypeStruct((M, H * 256), BF),
            jax.ShapeDtypeStruct((M, D), BF),
            jax.ShapeDtypeStruct((M, D), BF),
            jax.ShapeDtypeStruct((M, Dr), BF),
        ],
        scratch_shapes=[
            pltpu.VMEM((M, D), BF),
            pltpu.VMEM((M, DC), BF),
            pltpu.VMEM((DC, D), BF),
            pltpu.VMEM((DC, D), BF),
            pltpu.VMEM((M, DC), BF),
            pltpu.VMEM((DC, D), BF),
            pltpu.VMEM((DC, D), BF),
            pltpu.SemaphoreType.DMA((3,)),
            pltpu.SemaphoreType.DMA((3,)),
        ],
        compiler_params=pltpu.CompilerParams(
            collective_id=0, vmem_limit_bytes=60 * 1024 * 1024
        ),
    )(x, Wdkv, Wuk, Wuv, Wkr, Wq, Wqr)

    out = pl.pallas_call(
        _attn_o_body,
        grid=(B * NG,),
        in_specs=[
            pl.BlockSpec((S, GH * 256), lambda i: (i // NG, i % NG)),
            pl.BlockSpec((S, GH * Dh), lambda i: (i // NG, i % NG)),
            pl.BlockSpec((S, GH * Dh), lambda i: (i // NG, i % NG)),
            pl.BlockSpec((S, Dr), lambda i: (i // NG, 0)),
            pl.BlockSpec((D, D), lambda i: (0, 0)),
        ],
        out_specs=pl.BlockSpec((1, S, D), lambda i: (i // NG, 0, 0)),
        out_shape=jax.ShapeDtypeStruct((B, S, D), F32),
        scratch_shapes=[
            pltpu.VMEM((NG, S, GH * Dh), BF),
            pltpu.VMEM((D, D), BF),
        ],
        compiler_params=pltpu.CompilerParams(
            vmem_limit_bytes=60 * 1024 * 1024
        ),
    )(Q, K, V, Kr, Wo)
    return out


# device time: 68516 ns/iter; 1.0469x vs baseline; 1.0469x over previous
import jax
import jax.numpy as jnp
from jax import lax
from jax.experimental import pallas as pl
from jax.experimental.pallas import tpu as pltpu

B, S, D = 2, 512, 2048
H, Dh, Dr = 16, 128, 32
DC = 128
M = B * S
SCALE = (Dh + Dr) ** -0.5
BF = jnp.bfloat16
F32 = jnp.float32
NJ = 4


LOG2E = 1.4426950408889634


def _proj_comm_body(x_ref, wdkv_ref, wuk_ref, wuv_ref, wkr_ref, wq_ref, wqr_ref,
                    q_ref, k_ref, v_ref, kr_ref,
                    xb_s, c_s, wuk_s, wuv_s, c_r, wuk_r, wuv_r,
                    send_sems, recv_sems):
    j = pl.program_id(0)
    my_x = lax.axis_index("x")
    my_y = lax.axis_index("y")
    my_z = lax.axis_index("z")
    nbr = (1 - my_x, my_y, my_z)

    pairs = [(c_s, c_r), (wuk_s, wuk_r), (wuv_s, wuv_r)]

    def mk(i, src, dst):
        return pltpu.make_async_remote_copy(
            src_ref=src, dst_ref=dst,
            send_sem=send_sems.at[i], recv_sem=recv_sems.at[i],
            device_id=nbr, device_id_type=pl.DeviceIdType.MESH,
        )

    @pl.when(j == 0)
    def _():
        barrier = pltpu.get_barrier_semaphore()
        pl.semaphore_signal(barrier, inc=1, device_id=nbr,
                            device_id_type=pl.DeviceIdType.MESH)
        xb = x_ref[...].reshape(M, D).astype(BF)
        xb_s[...] = xb
        c_s[...] = jnp.dot(
            xb, wdkv_ref[...].astype(BF), preferred_element_type=F32
        ).astype(BF)
        wuk_s[...] = wuk_ref[...].astype(BF)
        wuv_s[...] = wuv_ref[...].astype(BF)
        pl.semaphore_wait(barrier, 1)
        for i, (s, d) in enumerate(pairs):
            mk(i, s, d).start()

    xb = xb_s[...]
    qd = (
        jnp.dot(xb, wq_ref[...].astype(BF), preferred_element_type=F32)
        * (SCALE * LOG2E)
    ).astype(BF)
    qrd = (
        jnp.dot(xb, wqr_ref[...].astype(BF), preferred_element_type=F32)
        * (SCALE * LOG2E)
    ).astype(BF)
    zpad = jnp.zeros((M, Dh - Dr), BF)
    for i in range(4):
        q_ref[:, i * 256:i * 256 + Dh] = qd[:, i * Dh:(i + 1) * Dh]
        q_ref[:, i * 256 + Dh:(i + 1) * 256] = jnp.concatenate(
            [qrd[:, i * Dr:(i + 1) * Dr], zpad], axis=1
        )

    @pl.when(j == NJ - 1)
    def _():
        kr_ref[...] = jnp.dot(
            xb, wkr_ref[...].astype(BF), preferred_element_type=F32
        ).astype(BF)
        for i, (s, d) in enumerate(pairs):
            mk(i, s, d).wait()
        ccat = jnp.concatenate([c_s[...], c_r[...]], axis=1)
        wukcat = jnp.concatenate([wuk_s[...], wuk_r[...]], axis=0)
        k_ref[...] = jnp.dot(
            ccat, wukcat, preferred_element_type=F32
        ).astype(BF)
        wuvcat = jnp.concatenate([wuv_s[...], wuv_r[...]], axis=0)
        v_ref[...] = jnp.dot(
            ccat, wuvcat, preferred_element_type=F32
        ).astype(BF)


def _attn_o_body(q_ref, k_ref, v_ref, kr_ref, wo_ref, out_ref,
                 o_s, wo_bf):
    b = pl.program_id(0)

    @pl.when(b == 0)
    def _():
        wo_bf[...] = wo_ref[...].astype(BF)

    nt = (((1,), (1,)), ((), ()))
    kr = kr_ref[...]
    for h in range(H):
        q = q_ref[:, h * 256:(h + 1) * 256]
        k = k_ref[:, h * Dh:(h + 1) * Dh]
        kcat = jnp.concatenate([k, kr, k[:, :Dh - Dr]], axis=1)
        s = lax.dot_general(q, kcat, nt, preferred_element_type=F32)
        p = jnp.exp2(s.astype(BF))
        r = 1.0 / jnp.sum(p.astype(F32), axis=-1, keepdims=True)
        o = jnp.dot(
            p, v_ref[:, h * Dh:(h + 1) * Dh],
            preferred_element_type=F32,
        )
        o_s[:, h * Dh:(h + 1) * Dh] = (o * r).astype(BF)

    out = jnp.dot(o_s[...], wo_bf[...], preferred_element_type=F32)
    out_ref[...] = out.reshape(1, S, D)


def kernel(x, Wdkv, Wuk, Wuv, Wq, Wqr, Wkr, Wo):
    BD = D // NJ
    BR = H * Dr // NJ

    Q, K, V, Kr = pl.pallas_call(
        _proj_comm_body,
        grid=(NJ,),
        in_specs=[
            pl.BlockSpec((B, S, D), lambda j: (0, 0, 0)),
            pl.BlockSpec((D, DC), lambda j: (0, 0)),
            pl.BlockSpec((DC, D), lambda j: (0, 0)),
            pl.BlockSpec((DC, D), lambda j: (0, 0)),
            pl.BlockSpec((D, Dr), lambda j: (0, 0)),
            pl.BlockSpec((D, BD), lambda j: (0, j)),
            pl.BlockSpec((D, BR), lambda j: (0, j)),
        ],
        out_specs=[
            pl.BlockSpec((M, 4 * 256), lambda j: (0, j)),
            pl.BlockSpec((M, D), lambda j: (0, 0)),
            pl.BlockSpec((M, D), lambda j: (0, 0)),
            pl.BlockSpec((M, Dr), lambda j: (0, 0)),
        ],
        out_shape=[
            jax.ShapeDtypeStruct((M, H * 256), BF),
            jax.ShapeDtypeStruct((M, D), BF),
            jax.ShapeDtypeStruct((M, D), BF),
            jax.ShapeDtypeStruct((M, Dr), BF),
        ],
        scratch_shapes=[
            pltpu.VMEM((M, D), BF),
            pltpu.VMEM((M, DC), BF),
            pltpu.VMEM((DC, D), BF),
            pltpu.VMEM((DC, D), BF),
            pltpu.VMEM((M, DC), BF),
            pltpu.VMEM((DC, D), BF),
            pltpu.VMEM((DC, D), BF),
            pltpu.SemaphoreType.DMA((3,)),
            pltpu.SemaphoreType.DMA((3,)),
        ],
        compiler_params=pltpu.CompilerParams(
            collective_id=0, vmem_limit_bytes=60 * 1024 * 1024
        ),
    )(x, Wdkv, Wuk, Wuv, Wkr, Wq, Wqr)

    out = pl.pallas_call(
        _attn_o_body,
        grid=(B,),
        in_specs=[
            pl.BlockSpec((S, H * 256), lambda b: (b, 0)),
            pl.BlockSpec((S, D), lambda b: (b, 0)),
            pl.BlockSpec((S, D), lambda b: (b, 0)),
            pl.BlockSpec((S, Dr), lambda b: (b, 0)),
            pl.BlockSpec((D, D), lambda b: (0, 0)),
        ],
        out_specs=pl.BlockSpec((1, S, D), lambda b: (b, 0, 0)),
        out_shape=jax.ShapeDtypeStruct((B, S, D), F32),
        scratch_shapes=[
            pltpu.VMEM((S, D), BF),
            pltpu.VMEM((D, D), BF),
        ],
        compiler_params=pltpu.CompilerParams(
            vmem_limit_bytes=60 * 1024 * 1024
        ),
    )(Q, K, V, Kr, Wo)
    return out
